# baseline (speedup 1.0000x reference)
import functools

import jax
import jax.numpy as jnp
from jax import lax
from jax.experimental import pallas as pl
from jax.experimental.pallas import tpu as pltpu
from jax.experimental.pallas import tpu_sc as plsc

D_SIZES = (10, 10, 20, 5, 10)
_COL_BASE = (0, 10, 20, 40, 45)
_PAD_D = (16, 16, 24, 16, 16)
D_SUM = 55
NUM_CODES = 100000
BATCH = 16384

_info = plsc.get_sparse_core_info()
_NC = _info.num_cores
_NS = _info.num_subcores
_NW = _NC * _NS
_BPW = BATCH // _NW
_CH = 128
_NCH = _BPW // _CH

_mesh = plsc.VectorSubcoreMesh(core_axis_name="c", subcore_axis_name="s")


@functools.partial(
    pl.kernel,
    mesh=_mesh,
    out_type=jax.ShapeDtypeStruct((D_SUM, BATCH), jnp.float32),
    scratch_types=[
        pltpu.VMEM((_NCH, _CH), jnp.int32),
        pltpu.VMEM((D_SUM, _BPW), jnp.float32),
        pltpu.SemaphoreType.DMA,
    ],
    compiler_params=pltpu.CompilerParams(use_tc_tiling_on_sc=False),
)
def _gather5(ids_hbm, w0t, w1t, w2t, w3t, w4t, out, idx_v, blk, sem):
    wid = lax.axis_index("s") * _NC + lax.axis_index("c")
    tabs = (w0t, w1t, w2t, w3t, w4t)

    pltpu.sync_copy(ids_hbm.at[pl.ds(wid * _NCH, _NCH)], idx_v)
    base = wid * _BPW

    copies = []
    for j in range(_NCH):
        idx_chunk = idx_v.at[j]
        k = 0
        for t, d in enumerate(D_SIZES):
            for f in range(d):
                copies.append(
                    pltpu.async_copy(
                        tabs[t].at[f].at[idx_chunk],
                        blk.at[k, pl.ds(j * _CH, _CH)], sem))
                k += 1
    for c in copies:
        c.wait()
    pltpu.sync_copy(blk, out.at[:, pl.ds(base, _BPW)])


def kernel(code_ids, W0, W1, W2, W3, W4):
    ids2d = code_ids.astype(jnp.int32).reshape(BATCH // _CH, _CH)
    packed = _gather5(ids2d, W0.T, W1.T, W2.T, W3.T, W4.T)
    return (packed, packed, packed, packed, packed)


# speedup vs baseline: 1.0332x; 1.0332x over previous
import functools

import jax
import jax.numpy as jnp
from jax import lax
from jax.experimental import pallas as pl
from jax.experimental.pallas import tpu as pltpu
from jax.experimental.pallas import tpu_sc as plsc

D_SIZES = (10, 10, 20, 5, 10)
_COL_BASE = (0, 10, 20, 40, 45)
_PAD_D = (16, 16, 24, 16, 16)
D_SUM = 55
NUM_CODES = 100000
BATCH = 16384

_info = plsc.get_sparse_core_info()
_NC = _info.num_cores
_NS = _info.num_subcores
_NW = _NC * _NS
_BPW = BATCH // _NW
_CH = 128
_NCH = _BPW // _CH

_mesh = plsc.VectorSubcoreMesh(core_axis_name="c", subcore_axis_name="s")


@functools.partial(
    pl.kernel,
    mesh=_mesh,
    out_type=jax.ShapeDtypeStruct((D_SUM, BATCH), jnp.float32),
    scratch_types=[
        pltpu.VMEM((_NCH, _CH), jnp.int32),
        pltpu.VMEM((D_SUM, _BPW), jnp.float32),
        pltpu.SemaphoreType.DMA,
    ],
    compiler_params=pltpu.CompilerParams(use_tc_tiling_on_sc=False),
)
def _gather5(ids_hbm, w0t, w1t, w2t, w3t, w4t, out, idx_v, blk, sem):
    wid = lax.axis_index("s") * _NC + lax.axis_index("c")
    tabs = (w0t, w1t, w2t, w3t, w4t)

    pltpu.sync_copy(ids_hbm.at[pl.ds(wid * _NCH, _NCH)], idx_v)
    base = wid * _BPW

    copies = []
    for j in range(_NCH):
        idx_chunk = idx_v.at[j]
        k = 0
        for t, d in enumerate(D_SIZES):
            for f in range(d):
                copies.append(
                    pltpu.async_copy(
                        tabs[t].at[f].at[idx_chunk],
                        blk.at[k, pl.ds(j * _CH, _CH)], sem))
                k += 1
    for c in copies:
        c.wait()
    pltpu.sync_copy(blk, out.at[:, pl.ds(base, _BPW)])


def kernel(code_ids, W0, W1, W2, W3, W4):
    ids2d = code_ids.astype(jnp.int32).reshape(BATCH // _CH, _CH)
    packed = _gather5(ids2d, W0.T, W1.T, W2.T, W3.T, W4.T)
    return tuple(
        packed[cb:cb + d, :].T for cb, d in zip(_COL_BASE, D_SIZES)
    )
